# baseline (speedup 1.0000x reference)
import functools

import jax
import jax.numpy as jnp
from jax import lax
from jax.experimental import pallas as pl
from jax.experimental.pallas import tpu as pltpu
from jax.experimental.pallas import tpu_sc as plsc

_NUM_CORES = 2
_NUM_SUBCORES = 16
_LANES = 16

_LOG2E = 1.4426950408889634


def _sc_xe(x, edge_types, mul_tbl, bias_tbl):
    e_total = x.shape[0]
    t = mul_tbl.shape[0]
    nw = _NUM_CORES * _NUM_SUBCORES
    chunk = e_total // nw
    assert e_total % nw == 0 and chunk % _LANES == 0 and chunk % 8 == 0

    mesh = plsc.VectorSubcoreMesh(
        core_axis_name="c", subcore_axis_name="s",
        num_cores=_NUM_CORES, num_subcores=_NUM_SUBCORES)

    @functools.partial(
        pl.kernel,
        out_type=jax.ShapeDtypeStruct((e_total,), jnp.float32),
        mesh=mesh,
        scratch_types=[
            pltpu.VMEM((chunk,), jnp.int32),
            pltpu.VMEM((chunk,), jnp.float32),
            pltpu.VMEM((chunk,), jnp.float32),
            pltpu.VMEM((t,), jnp.float32),
            pltpu.VMEM((t,), jnp.float32),
            pltpu.SemaphoreType.DMA,
            pltpu.SemaphoreType.DMA,
            pltpu.SemaphoreType.DMA,
            pltpu.SemaphoreType.DMA,
        ],
        compiler_params=pltpu.CompilerParams(needs_layout_passes=False),
    )
    def body(x_hbm, et_hbm, mul_hbm, bias_hbm, xe_hbm,
             idx_v, x_v, xe_v, mul_v, bias_v, sem0, sem1, sem2, sem3):
        wid = lax.axis_index("s") * _NUM_CORES + lax.axis_index("c")
        base = wid * chunk
        c0 = pltpu.async_copy(mul_hbm, mul_v, sem0)
        c1 = pltpu.async_copy(bias_hbm, bias_v, sem1)
        c2 = pltpu.async_copy(et_hbm.at[pl.ds(base, chunk)], idx_v, sem2)
        c3 = pltpu.async_copy(x_hbm.at[pl.ds(base, chunk)], x_v, sem3)
        c0.wait()
        c1.wait()
        c2.wait()
        c3.wait()

        @plsc.parallel_loop(0, chunk // _LANES, unroll=8)
        def step(i):
            s = pl.ds(i * _LANES, _LANES)
            idx = idx_v[s]
            m = plsc.load_gather(mul_v, [idx])
            b = plsc.load_gather(bias_v, [idx])
            xe_v[s] = m * x_v[s] + b

        pltpu.sync_copy(xe_v, xe_hbm.at[pl.ds(base, chunk)])

    return body(x, edge_types, mul_tbl, bias_tbl)


def _tc_rbf_body(means_ref, temps_ref, xe_ref, out_ref):
    m = means_ref[...]
    nt = jnp.abs(temps_ref[...]) * (-_LOG2E)
    xe = xe_ref[0]
    hi = xe.astype(jnp.bfloat16).astype(jnp.float32)
    r1 = xe - hi
    mid = r1.astype(jnp.bfloat16).astype(jnp.float32)
    lo = r1 - mid
    lhs = jnp.concatenate([hi, mid, lo], axis=0)
    ones3 = jnp.ones((3, m.shape[1]), jnp.float32)
    xe_b = lax.dot_general(lhs, ones3, (((0,), (0,)), ((), ())),
                           preferred_element_type=jnp.float32)
    d = xe_b - m
    out_ref[...] = jnp.exp2(d * d * nt)


def _tc_rbf_part(xe, means, temps, block_e, e_total, row_offset, prev=None):
    e_part = xe.shape[0]
    k = means.shape[0]
    assert e_part % block_e == 0 and row_offset % block_e == 0
    blk_off = row_offset // block_e
    grid = (e_part // block_e,)
    in_specs = [
        pl.BlockSpec((1, k), lambda i: (0, 0)),
        pl.BlockSpec((1, k), lambda i: (0, 0)),
        pl.BlockSpec((1, 1, block_e), lambda i: (i, 0, 0)),
    ]
    args = [means.reshape(1, k), temps.reshape(1, k),
            xe.reshape(e_part // block_e, 1, block_e)]
    aliases = {}
    if prev is not None:
        in_specs.append(pl.BlockSpec(memory_space=pl.ANY))
        args.append(prev)
        aliases = {3: 0}

    def body(means_ref, temps_ref, xe_ref, *rest):
        out_ref = rest[-1]
        _tc_rbf_body(means_ref, temps_ref, xe_ref, out_ref)

    return pl.pallas_call(
        body,
        grid=grid,
        in_specs=in_specs,
        out_specs=pl.BlockSpec((block_e, k), lambda i: (i + blk_off, 0)),
        out_shape=jax.ShapeDtypeStruct((e_total, k), jnp.float32),
        input_output_aliases=aliases,
        compiler_params=pltpu.CompilerParams(
            dimension_semantics=("arbitrary",),
        ),
    )(*args)


def kernel(x, edge_types, means, temps, mul_weight, bias_weight):
    e_total = x.shape[0]
    et = edge_types.astype(jnp.int32)
    xf = x.astype(jnp.float32)
    means_f = means.astype(jnp.float32)
    temps_f = temps.astype(jnp.float32)
    mul_tbl = mul_weight.reshape(-1).astype(jnp.float32)
    bias_tbl = bias_weight.reshape(-1).astype(jnp.float32)

    e_a = 64000
    xe_a = _sc_xe(xf[:e_a], et[:e_a], mul_tbl, bias_tbl)
    xe_b = _sc_xe(xf[e_a:], et[e_a:], mul_tbl, bias_tbl)
    out_a = _tc_rbf_part(xe_a, means_f, temps_f, block_e=8000,
                         e_total=e_total, row_offset=0)
    out = _tc_rbf_part(xe_b, means_f, temps_f, block_e=32000,
                       e_total=e_total, row_offset=e_a, prev=out_a)
    return out.astype(means.dtype)


# speedup vs baseline: 1.0407x; 1.0407x over previous
import functools

import jax
import jax.numpy as jnp
from jax import lax
from jax.experimental import pallas as pl
from jax.experimental.pallas import tpu as pltpu
from jax.experimental.pallas import tpu_sc as plsc

_NUM_CORES = 2
_NUM_SUBCORES = 16
_LANES = 16

_LOG2E = 1.4426950408889634


def _sc_xe(x, edge_types, mul_tbl, bias_tbl):
    e_total = x.shape[0]
    t = mul_tbl.shape[0]
    nw = _NUM_CORES * _NUM_SUBCORES
    chunk = e_total // nw
    assert e_total % nw == 0 and chunk % _LANES == 0 and chunk % 8 == 0

    mesh = plsc.VectorSubcoreMesh(
        core_axis_name="c", subcore_axis_name="s",
        num_cores=_NUM_CORES, num_subcores=_NUM_SUBCORES)

    @functools.partial(
        pl.kernel,
        out_type=jax.ShapeDtypeStruct((e_total,), jnp.float32),
        mesh=mesh,
        scratch_types=[
            pltpu.VMEM((chunk,), jnp.int32),
            pltpu.VMEM((chunk,), jnp.float32),
            pltpu.VMEM((chunk,), jnp.float32),
            pltpu.VMEM((t,), jnp.float32),
            pltpu.VMEM((t,), jnp.float32),
            pltpu.SemaphoreType.DMA,
            pltpu.SemaphoreType.DMA,
            pltpu.SemaphoreType.DMA,
            pltpu.SemaphoreType.DMA,
        ],
        compiler_params=pltpu.CompilerParams(needs_layout_passes=False),
    )
    def body(x_hbm, et_hbm, mul_hbm, bias_hbm, xe_hbm,
             idx_v, x_v, xe_v, mul_v, bias_v, sem0, sem1, sem2, sem3):
        wid = lax.axis_index("s") * _NUM_CORES + lax.axis_index("c")
        base = wid * chunk
        c0 = pltpu.async_copy(mul_hbm, mul_v, sem0)
        c1 = pltpu.async_copy(bias_hbm, bias_v, sem1)
        c2 = pltpu.async_copy(et_hbm.at[pl.ds(base, chunk)], idx_v, sem2)
        c3 = pltpu.async_copy(x_hbm.at[pl.ds(base, chunk)], x_v, sem3)
        c0.wait()
        c1.wait()
        c2.wait()
        c3.wait()

        @plsc.parallel_loop(0, chunk // _LANES, unroll=8)
        def step(i):
            s = pl.ds(i * _LANES, _LANES)
            idx = idx_v[s]
            m = plsc.load_gather(mul_v, [idx])
            b = plsc.load_gather(bias_v, [idx])
            xe_v[s] = m * x_v[s] + b

        pltpu.sync_copy(xe_v, xe_hbm.at[pl.ds(base, chunk)])

    return body(x, edge_types, mul_tbl, bias_tbl)


def _tc_rbf_body(means_ref, temps_ref, xe_ref, out_ref):
    m = means_ref[...]
    nt = jnp.abs(temps_ref[...]) * (-_LOG2E)
    xe = xe_ref[0]
    hi = xe.astype(jnp.bfloat16).astype(jnp.float32)
    r1 = xe - hi
    mid = r1.astype(jnp.bfloat16).astype(jnp.float32)
    lo = r1 - mid
    lhs = jnp.concatenate([hi, mid, lo], axis=0)
    ones3 = jnp.ones((3, m.shape[1]), jnp.float32)
    xe_b = lax.dot_general(lhs, ones3, (((0,), (0,)), ((), ())),
                           preferred_element_type=jnp.float32)
    d = xe_b - m
    out_ref[...] = jnp.exp2(d * d * nt)


def _tc_rbf_part(xe, means, temps, block_e, e_total, row_offset, prev=None):
    e_part = xe.shape[0]
    k = means.shape[0]
    assert e_part % block_e == 0 and row_offset % block_e == 0
    blk_off = row_offset // block_e
    grid = (e_part // block_e,)
    in_specs = [
        pl.BlockSpec((1, k), lambda i: (0, 0)),
        pl.BlockSpec((1, k), lambda i: (0, 0)),
        pl.BlockSpec((1, 1, block_e), lambda i: (i, 0, 0)),
    ]
    args = [means.reshape(1, k), temps.reshape(1, k),
            xe.reshape(e_part // block_e, 1, block_e)]
    aliases = {}
    if prev is not None:
        in_specs.append(pl.BlockSpec(memory_space=pl.ANY))
        args.append(prev)
        aliases = {3: 0}

    def body(means_ref, temps_ref, xe_ref, *rest):
        out_ref = rest[-1]
        _tc_rbf_body(means_ref, temps_ref, xe_ref, out_ref)

    return pl.pallas_call(
        body,
        grid=grid,
        in_specs=in_specs,
        out_specs=pl.BlockSpec((block_e, k), lambda i: (i + blk_off, 0)),
        out_shape=jax.ShapeDtypeStruct((e_total, k), jnp.float32),
        input_output_aliases=aliases,
        compiler_params=pltpu.CompilerParams(
            dimension_semantics=("arbitrary",),
        ),
    )(*args)


def _tc_rbf_manual(xe, means, temps, block_e):
    e_total = xe.shape[0]
    k = means.shape[0]
    assert e_total % block_e == 0
    n = e_total // block_e
    assert n >= 2

    def body(means_ref, temps_ref, xe_ref, out_hbm, buf, sems):
        i = pl.program_id(0)
        slot = lax.rem(i, 2)

        @pl.when(i >= 2)
        def _wait_prev():
            pltpu.make_async_copy(
                buf.at[slot],
                out_hbm.at[pl.ds((i - 2) * block_e, block_e), :],
                sems.at[slot]).wait()

        m = means_ref[...]
        nt = jnp.abs(temps_ref[...]) * (-_LOG2E)
        xe_v = xe_ref[0]
        hi = xe_v.astype(jnp.bfloat16).astype(jnp.float32)
        r1 = xe_v - hi
        mid = r1.astype(jnp.bfloat16).astype(jnp.float32)
        lo = r1 - mid
        lhs = jnp.concatenate([hi, mid, lo], axis=0)
        ones3 = jnp.ones((3, k), jnp.float32)
        xe_b = lax.dot_general(lhs, ones3, (((0,), (0,)), ((), ())),
                               preferred_element_type=jnp.float32)
        d = xe_b - m
        buf[slot] = jnp.exp2(d * d * nt)

        pltpu.async_copy(
            buf.at[slot],
            out_hbm.at[pl.ds(i * block_e, block_e), :],
            sems.at[slot])

        @pl.when(i == n - 1)
        def _drain():
            other = 1 - slot
            pltpu.make_async_copy(
                buf.at[other],
                out_hbm.at[pl.ds((i - 1) * block_e, block_e), :],
                sems.at[other]).wait()
            pltpu.make_async_copy(
                buf.at[slot],
                out_hbm.at[pl.ds(i * block_e, block_e), :],
                sems.at[slot]).wait()

    return pl.pallas_call(
        body,
        grid=(n,),
        in_specs=[
            pl.BlockSpec((1, k), lambda i: (0, 0)),
            pl.BlockSpec((1, k), lambda i: (0, 0)),
            pl.BlockSpec((1, 1, block_e), lambda i: (i, 0, 0)),
        ],
        out_specs=pl.BlockSpec(memory_space=pl.ANY),
        out_shape=jax.ShapeDtypeStruct((e_total, k), jnp.float32),
        scratch_shapes=[
            pltpu.VMEM((2, block_e, k), jnp.float32),
            pltpu.SemaphoreType.DMA((2,)),
        ],
        compiler_params=pltpu.CompilerParams(
            dimension_semantics=("arbitrary",),
        ),
    )(means.reshape(1, k), temps.reshape(1, k),
      xe.reshape(n, 1, block_e))


def kernel(x, edge_types, means, temps, mul_weight, bias_weight):
    e_total = x.shape[0]
    et = edge_types.astype(jnp.int32)
    xf = x.astype(jnp.float32)
    means_f = means.astype(jnp.float32)
    temps_f = temps.astype(jnp.float32)
    mul_tbl = mul_weight.reshape(-1).astype(jnp.float32)
    bias_tbl = bias_weight.reshape(-1).astype(jnp.float32)

    xe = _sc_xe(xf, et, mul_tbl, bias_tbl)
    out = _tc_rbf_manual(xe, means_f, temps_f, block_e=32000)
    return out.astype(means.dtype)


# speedup vs baseline: 1.0426x; 1.0017x over previous
import functools

import jax
import jax.numpy as jnp
from jax import lax
from jax.experimental import pallas as pl
from jax.experimental.pallas import tpu as pltpu
from jax.experimental.pallas import tpu_sc as plsc

_NUM_CORES = 2
_NUM_SUBCORES = 16
_LANES = 16

_LOG2E = 1.4426950408889634


def _sc_xe(x, edge_types, mul_tbl, bias_tbl):
    e_total = x.shape[0]
    t = mul_tbl.shape[0]
    nw = _NUM_CORES * _NUM_SUBCORES
    chunk = e_total // nw
    assert e_total % nw == 0 and chunk % _LANES == 0 and chunk % 8 == 0

    mesh = plsc.VectorSubcoreMesh(
        core_axis_name="c", subcore_axis_name="s",
        num_cores=_NUM_CORES, num_subcores=_NUM_SUBCORES)

    @functools.partial(
        pl.kernel,
        out_type=jax.ShapeDtypeStruct((e_total,), jnp.float32),
        mesh=mesh,
        scratch_types=[
            pltpu.VMEM((chunk,), jnp.int32),
            pltpu.VMEM((chunk,), jnp.float32),
            pltpu.VMEM((chunk,), jnp.float32),
            pltpu.VMEM((t,), jnp.float32),
            pltpu.VMEM((t,), jnp.float32),
            pltpu.SemaphoreType.DMA,
            pltpu.SemaphoreType.DMA,
            pltpu.SemaphoreType.DMA,
            pltpu.SemaphoreType.DMA,
        ],
        compiler_params=pltpu.CompilerParams(needs_layout_passes=False),
    )
    def body(x_hbm, et_hbm, mul_hbm, bias_hbm, xe_hbm,
             idx_v, x_v, xe_v, mul_v, bias_v, sem0, sem1, sem2, sem3):
        wid = lax.axis_index("s") * _NUM_CORES + lax.axis_index("c")
        base = wid * chunk
        c0 = pltpu.async_copy(mul_hbm, mul_v, sem0)
        c1 = pltpu.async_copy(bias_hbm, bias_v, sem1)
        c2 = pltpu.async_copy(et_hbm.at[pl.ds(base, chunk)], idx_v, sem2)
        c3 = pltpu.async_copy(x_hbm.at[pl.ds(base, chunk)], x_v, sem3)
        c0.wait()
        c1.wait()
        c2.wait()
        c3.wait()

        @plsc.parallel_loop(0, chunk // _LANES, unroll=8)
        def step(i):
            s = pl.ds(i * _LANES, _LANES)
            idx = idx_v[s]
            m = plsc.load_gather(mul_v, [idx])
            b = plsc.load_gather(bias_v, [idx])
            xe_v[s] = m * x_v[s] + b

        pltpu.sync_copy(xe_v, xe_hbm.at[pl.ds(base, chunk)])

    return body(x, edge_types, mul_tbl, bias_tbl)


def _tc_rbf_body(means_ref, temps_ref, xe_ref, out_ref):
    m = means_ref[...]
    nt = jnp.abs(temps_ref[...]) * (-_LOG2E)
    xe = xe_ref[0]
    hi = xe.astype(jnp.bfloat16).astype(jnp.float32)
    r1 = xe - hi
    mid = r1.astype(jnp.bfloat16).astype(jnp.float32)
    lo = r1 - mid
    lhs = jnp.concatenate([hi, mid, lo], axis=0)
    ones3 = jnp.ones((3, m.shape[1]), jnp.float32)
    xe_b = lax.dot_general(lhs, ones3, (((0,), (0,)), ((), ())),
                           preferred_element_type=jnp.float32)
    d = xe_b - m
    out_ref[...] = jnp.exp2(d * d * nt)


def _tc_rbf(xe, means, temps, block_e):
    e_total = xe.shape[0]
    k = means.shape[0]
    assert e_total % block_e == 0
    grid = (e_total // block_e,)
    return pl.pallas_call(
        _tc_rbf_body,
        grid=grid,
        in_specs=[
            pl.BlockSpec((1, k), lambda i: (0, 0)),
            pl.BlockSpec((1, k), lambda i: (0, 0)),
            pl.BlockSpec((1, 1, block_e), lambda i: (i, 0, 0)),
        ],
        out_specs=pl.BlockSpec((block_e, k), lambda i: (i, 0)),
        out_shape=jax.ShapeDtypeStruct((e_total, k), jnp.float32),
        compiler_params=pltpu.CompilerParams(
            dimension_semantics=("arbitrary",),
        ),
    )(means.reshape(1, k), temps.reshape(1, k),
      xe.reshape(e_total // block_e, 1, block_e))


def kernel(x, edge_types, means, temps, mul_weight, bias_weight):
    et = edge_types.astype(jnp.int32)
    mul_tbl = mul_weight.reshape(-1).astype(jnp.float32)
    bias_tbl = bias_weight.reshape(-1).astype(jnp.float32)
    xe = _sc_xe(x.astype(jnp.float32), et, mul_tbl, bias_tbl)
    out = _tc_rbf(xe, means.astype(jnp.float32), temps.astype(jnp.float32),
                  block_e=32000)
    return out.astype(means.dtype)
